# Initial kernel scaffold; baseline (speedup 1.0000x reference)
import jax
import jax.numpy as jnp
from jax.experimental import pallas as pl


def kernel(xyz, new_xyz, features):
    raise NotImplementedError("write your pallas kernel here")


# speedup vs baseline: 7.0497x; 7.0497x over previous
import functools

import jax
import jax.numpy as jnp
from jax import lax
from jax.experimental import pallas as pl
from jax.experimental.pallas import tpu as pltpu
from jax.experimental.pallas import tpu_sc as plsc

_INTERPRET = False

NSAMPLE = 32
QB = 256


def _knn_body(q_ref, xt_ref, idx_ref):
    q = q_ref[0]
    xt = xt_ref[0]
    n = xt.shape[1]
    qb = q.astype(jnp.bfloat16).astype(jnp.float32)
    xb = xt.astype(jnp.bfloat16).astype(jnp.float32)
    ab = ((qb[:, 0:1] * xb[0:1, :] + qb[:, 1:2] * xb[1:2, :])
          + qb[:, 2:3] * xb[2:3, :])
    a2 = (q[:, 0:1] * q[:, 0:1] + q[:, 1:2] * q[:, 1:2]) + q[:, 2:3] * q[:, 2:3]
    b2 = (xt[0:1, :] * xt[0:1, :] + xt[1:2, :] * xt[1:2, :]) + xt[2:3, :] * xt[2:3, :]
    d2 = a2 - 2.0 * ab + b2
    iota = lax.broadcasted_iota(jnp.int32, d2.shape, 1)
    inf = jnp.float32(jnp.inf)
    big = jnp.int32(n)
    for k in range(NSAMPLE):
        m = jnp.min(d2, axis=1, keepdims=True)
        cand = jnp.where(d2 == m, iota, big)
        amin = jnp.min(cand, axis=1, keepdims=True)
        idx_ref[0, :, k:k + 1] = amin
        d2 = jnp.where(cand == amin, inf, d2)


def _knn(qpad, xtpad):
    B, M, _ = qpad.shape
    N = xtpad.shape[2]
    return pl.pallas_call(
        _knn_body,
        grid=(B, M // QB),
        in_specs=[
            pl.BlockSpec((1, QB, 8), lambda b, m: (b, m, 0)),
            pl.BlockSpec((1, 8, N), lambda b, m: (b, 0, 0)),
        ],
        out_specs=pl.BlockSpec((1, QB, NSAMPLE), lambda b, m: (b, m, 0)),
        out_shape=jax.ShapeDtypeStruct((B, M, NSAMPLE), jnp.int32),
        interpret=_INTERPRET,
    )(qpad, xtpad)


def _make_gather(B, C, N, M):
    MK = M * NSAMPLE
    CHUNK = min(16384, MK)
    NCH = C // 16
    mesh = plsc.VectorSubcoreMesh(
        core_axis_name="c", subcore_axis_name="s", num_cores=2,
        num_subcores=16)

    @functools.partial(
        pl.kernel,
        out_type=jax.ShapeDtypeStruct((B, C + 3, MK), jnp.float32),
        mesh=mesh,
        scratch_types=[
            pltpu.VMEM((MK,), jnp.int32),
            pltpu.VMEM((N,), jnp.float32),
            pltpu.VMEM((CHUNK,), jnp.float32),
            pltpu.VMEM((M,), jnp.float32),
            pltpu.VMEM((N,), jnp.float32),
        ],
        compiler_params=pltpu.CompilerParams(needs_layout_passes=False),
        interpret=_INTERPRET,
    )
    def gather_kernel(feat_hbm, xyzt_hbm, nxt_hbm, idx_hbm, out_hbm,
                      idx_v, tab_v, out_v, nx_v, xtab_v):
        wid = lax.axis_index("s") * 2 + lax.axis_index("c")
        b = wid // 16
        j = wid % 16
        pltpu.sync_copy(idx_hbm.at[b], idx_v)

        def gather_chunks(cc, subtract, tab_v):
            for ch in range(MK // CHUNK):
                def body(i, _, ch=ch):
                    base = i * 128
                    for u in range(8):
                        off = base + u * 16
                        iv = idx_v[pl.ds(ch * CHUNK + off, 16)]
                        v = plsc.load_gather(tab_v, [iv])
                        if subtract:
                            pos = (jnp.int32(ch * CHUNK) + off
                                   + lax.broadcasted_iota(jnp.int32, (16,), 0))
                            mv = lax.shift_right_logical(pos, 5)
                            s = plsc.load_gather(nx_v, [mv])
                            v = v - s
                        out_v[pl.ds(off, 16)] = v
                    return 0
                lax.fori_loop(0, CHUNK // 128, body, 0)
                pltpu.sync_copy(
                    out_v, out_hbm.at[b, cc, pl.ds(ch * CHUNK, CHUNK)])

        @pl.when(j < 3)
        def _():
            pltpu.sync_copy(xyzt_hbm.at[b, j], xtab_v)
            pltpu.sync_copy(nxt_hbm.at[b, j], nx_v)
            gather_chunks(j, True, xtab_v)

        for t in range(NCH):
            c = j * NCH + t
            pltpu.sync_copy(feat_hbm.at[b, c], tab_v)
            gather_chunks(3 + c, False, tab_v)

    return gather_kernel


def kernel(xyz, new_xyz, features):
    B, N, _ = xyz.shape
    M = new_xyz.shape[1]
    C = features.shape[1]
    pad_q = jnp.zeros((B, M, 5), jnp.float32)
    qpad = jnp.concatenate([new_xyz, pad_q], axis=-1)
    xt = jnp.transpose(xyz, (0, 2, 1))
    pad_x = jnp.zeros((B, 5, N), jnp.float32)
    xtpad = jnp.concatenate([xt, pad_x], axis=1)
    idx = _knn(qpad, xtpad)
    nxt = jnp.transpose(new_xyz, (0, 2, 1))
    idxf = idx.reshape(B, M * NSAMPLE)
    out = _make_gather(B, C, N, M)(features, xt, nxt, idxf)
    return out.reshape(B, C + 3, M, NSAMPLE)


# speedup vs baseline: 7.2566x; 1.0293x over previous
import functools

import jax
import jax.numpy as jnp
from jax import lax
from jax.experimental import pallas as pl
from jax.experimental.pallas import tpu as pltpu
from jax.experimental.pallas import tpu_sc as plsc

_INTERPRET = False

NSAMPLE = 32
QB = 256


def _knn_body(q_ref, xt_ref, idx_ref):
    q = q_ref[0]
    xt = xt_ref[0]
    n = xt.shape[1]
    qb = q.astype(jnp.bfloat16).astype(jnp.float32)
    xb = xt.astype(jnp.bfloat16).astype(jnp.float32)
    ab = ((qb[:, 0:1] * xb[0:1, :] + qb[:, 1:2] * xb[1:2, :])
          + qb[:, 2:3] * xb[2:3, :])
    a2 = (q[:, 0:1] * q[:, 0:1] + q[:, 1:2] * q[:, 1:2]) + q[:, 2:3] * q[:, 2:3]
    b2 = (xt[0:1, :] * xt[0:1, :] + xt[1:2, :] * xt[1:2, :]) + xt[2:3, :] * xt[2:3, :]
    d2 = a2 - 2.0 * ab + b2
    iota = lax.broadcasted_iota(jnp.int32, d2.shape, 1)
    inf = jnp.float32(jnp.inf)
    big = jnp.int32(n)
    for k in range(NSAMPLE):
        m = jnp.min(d2, axis=1, keepdims=True)
        cand = jnp.where(d2 == m, iota, big)
        amin = jnp.min(cand, axis=1, keepdims=True)
        idx_ref[0, :, k:k + 1] = amin
        d2 = jnp.where(cand == amin, inf, d2)


def _knn(qpad, xtpad):
    B, M, _ = qpad.shape
    N = xtpad.shape[2]
    return pl.pallas_call(
        _knn_body,
        grid=(B, M // QB),
        in_specs=[
            pl.BlockSpec((1, QB, 8), lambda b, m: (b, m, 0)),
            pl.BlockSpec((1, 8, N), lambda b, m: (b, 0, 0)),
        ],
        out_specs=pl.BlockSpec((1, QB, NSAMPLE), lambda b, m: (b, m, 0)),
        out_shape=jax.ShapeDtypeStruct((B, M, NSAMPLE), jnp.int32),
        interpret=_INTERPRET,
    )(qpad, xtpad)


def _make_gather(B, C, N, M):
    K = NSAMPLE
    MK = M * K
    CHUNK = min(16384, MK)
    ROWS = CHUNK // K
    NCH = C // 16
    mesh = plsc.VectorSubcoreMesh(
        core_axis_name="c", subcore_axis_name="s", num_cores=2,
        num_subcores=16)

    @functools.partial(
        pl.kernel,
        out_type=jax.ShapeDtypeStruct((B, C + 3, MK), jnp.float32),
        mesh=mesh,
        scratch_types=[
            pltpu.VMEM((MK,), jnp.int32),
            pltpu.VMEM((N,), jnp.float32),
            pltpu.VMEM((N,), jnp.float32),
            pltpu.VMEM((CHUNK,), jnp.float32),
            pltpu.VMEM((CHUNK,), jnp.float32),
            pltpu.VMEM((M,), jnp.float32),
            pltpu.VMEM((N,), jnp.float32),
            pltpu.SemaphoreType.DMA,
            pltpu.SemaphoreType.DMA,
            pltpu.SemaphoreType.DMA,
        ],
        compiler_params=pltpu.CompilerParams(needs_layout_passes=False),
        interpret=_INTERPRET,
    )
    def gather_kernel(feat_hbm, xyzt_hbm, nxt_hbm, idx_hbm, out_hbm,
                      idx_v, tab0_v, tab1_v, out0_v, out1_v, nx_v, xtab_v,
                      sem0, sem1, tsem):
        osems = (sem0, sem1)
        tabs = (tab0_v, tab1_v)
        outs = (out0_v, out1_v)
        wid = lax.axis_index("s") * 2 + lax.axis_index("c")
        b = wid // 16
        j = wid % 16
        pltpu.sync_copy(idx_hbm.at[b], idx_v)
        pending = [None, None]

        def gather_channel(cc, tab, subtract):
            for ch in range(MK // CHUNK):
                s = ch % 2
                if pending[s] is not None:
                    pending[s].wait()
                buf = outs[s]

                def body(i, _, ch=ch, buf=buf, tab=tab):
                    for u in range(8):
                        off = i * 128 + u * 16
                        iv = idx_v[pl.ds(ch * CHUNK + off, 16)]
                        v = plsc.load_gather(tab, [iv])
                        if subtract:
                            pos = (jnp.int32(ch * CHUNK) + off
                                   + lax.broadcasted_iota(jnp.int32, (16,), 0))
                            mv = lax.shift_right_logical(pos, 5)
                            v = v - plsc.load_gather(nx_v, [mv])
                        buf[pl.ds(off, 16)] = v
                    return 0
                lax.fori_loop(0, CHUNK // 128, body, 0)
                pending[s] = pltpu.async_copy(
                    buf, out_hbm.at[b, cc, pl.ds(ch * CHUNK, CHUNK)],
                    osems[s])

        @pl.when(j < 3)
        def _():
            pltpu.sync_copy(xyzt_hbm.at[b, j], xtab_v)
            pltpu.sync_copy(nxt_hbm.at[b, j], nx_v)
            gather_channel(j, xtab_v, True)
            for s in range(2):
                if pending[s] is not None:
                    pending[s].wait()
                    pending[s] = None

        pltpu.sync_copy(feat_hbm.at[b, j * NCH], tabs[0])
        for t in range(NCH):
            c = j * NCH + t
            if t + 1 < NCH:
                tcp = pltpu.async_copy(
                    feat_hbm.at[b, c + 1], tabs[(t + 1) % 2], tsem)
            gather_channel(3 + c, tabs[t % 2], False)
            if t + 1 < NCH:
                tcp.wait()
        for p in pending:
            if p is not None:
                p.wait()

    return gather_kernel


def kernel(xyz, new_xyz, features):
    B, N, _ = xyz.shape
    M = new_xyz.shape[1]
    C = features.shape[1]
    pad_q = jnp.zeros((B, M, 5), jnp.float32)
    qpad = jnp.concatenate([new_xyz, pad_q], axis=-1)
    xt = jnp.transpose(xyz, (0, 2, 1))
    pad_x = jnp.zeros((B, 5, N), jnp.float32)
    xtpad = jnp.concatenate([xt, pad_x], axis=1)
    idx = _knn(qpad, xtpad)
    nxt = jnp.transpose(new_xyz, (0, 2, 1))
    idxf = idx.reshape(B, M * NSAMPLE)
    out = _make_gather(B, C, N, M)(features, xt, nxt, idxf)
    return out.reshape(B, C + 3, M, NSAMPLE)


# speedup vs baseline: 7.7149x; 1.0632x over previous
import functools

import jax
import jax.numpy as jnp
from jax import lax
from jax.experimental import pallas as pl
from jax.experimental.pallas import tpu as pltpu
from jax.experimental.pallas import tpu_sc as plsc

_INTERPRET = False

NSAMPLE = 32
QB = 512


def _knn_body(q_ref, xt_ref, idx_ref):
    q = q_ref[0]
    xt = xt_ref[0]
    n = xt.shape[1]
    qb = q.astype(jnp.bfloat16).astype(jnp.float32)
    xb = xt.astype(jnp.bfloat16).astype(jnp.float32)
    ab = ((qb[:, 0:1] * xb[0:1, :] + qb[:, 1:2] * xb[1:2, :])
          + qb[:, 2:3] * xb[2:3, :])
    a2 = (q[:, 0:1] * q[:, 0:1] + q[:, 1:2] * q[:, 1:2]) + q[:, 2:3] * q[:, 2:3]
    b2 = (xt[0:1, :] * xt[0:1, :] + xt[1:2, :] * xt[1:2, :]) + xt[2:3, :] * xt[2:3, :]
    d2 = a2 - 2.0 * ab + b2
    iota = lax.broadcasted_iota(jnp.int32, d2.shape, 1)
    inf = jnp.float32(jnp.inf)
    big = jnp.int32(n)
    for k in range(NSAMPLE):
        m = jnp.min(d2, axis=1, keepdims=True)
        cand = jnp.where(d2 == m, iota, big)
        amin = jnp.min(cand, axis=1, keepdims=True)
        idx_ref[0, :, k:k + 1] = amin
        d2 = jnp.where(cand == amin, inf, d2)


def _knn(qpad, xtpad):
    B, M, _ = qpad.shape
    N = xtpad.shape[2]
    return pl.pallas_call(
        _knn_body,
        grid=(B, M // QB),
        in_specs=[
            pl.BlockSpec((1, QB, 8), lambda b, m: (b, m, 0)),
            pl.BlockSpec((1, 8, N), lambda b, m: (b, 0, 0)),
        ],
        out_specs=pl.BlockSpec((1, QB, NSAMPLE), lambda b, m: (b, m, 0)),
        out_shape=jax.ShapeDtypeStruct((B, M, NSAMPLE), jnp.int32),
        interpret=_INTERPRET,
    )(qpad, xtpad)


def _make_gather(B, C, N, M):
    K = NSAMPLE
    MK = M * K
    CHUNK = min(16384, MK)
    ROWS = CHUNK // K
    NCH = C // 16
    mesh = plsc.VectorSubcoreMesh(
        core_axis_name="c", subcore_axis_name="s", num_cores=2,
        num_subcores=16)

    @functools.partial(
        pl.kernel,
        out_type=jax.ShapeDtypeStruct((B, C + 3, MK), jnp.float32),
        mesh=mesh,
        scratch_types=[
            pltpu.VMEM((MK,), jnp.int32),
            pltpu.VMEM((N,), jnp.float32),
            pltpu.VMEM((N,), jnp.float32),
            pltpu.VMEM((CHUNK,), jnp.float32),
            pltpu.VMEM((CHUNK,), jnp.float32),
            pltpu.VMEM((M,), jnp.float32),
            pltpu.VMEM((N,), jnp.float32),
            pltpu.SemaphoreType.DMA,
            pltpu.SemaphoreType.DMA,
            pltpu.SemaphoreType.DMA,
        ],
        compiler_params=pltpu.CompilerParams(needs_layout_passes=False),
        interpret=_INTERPRET,
    )
    def gather_kernel(feat_hbm, xyzt_hbm, nxt_hbm, idx_hbm, out_hbm,
                      idx_v, tab0_v, tab1_v, out0_v, out1_v, nx_v, xtab_v,
                      sem0, sem1, tsem):
        osems = (sem0, sem1)
        tabs = (tab0_v, tab1_v)
        outs = (out0_v, out1_v)
        wid = lax.axis_index("s") * 2 + lax.axis_index("c")
        b = wid // 16
        j = wid % 16
        pltpu.sync_copy(idx_hbm.at[b], idx_v)
        pending = [None, None]

        def gather_channel(cc, tab, subtract):
            for ch in range(MK // CHUNK):
                s = ch % 2
                if pending[s] is not None:
                    pending[s].wait()
                buf = outs[s]

                def body(i, _, ch=ch, buf=buf, tab=tab):
                    for u in range(8):
                        off = i * 128 + u * 16
                        iv = idx_v[pl.ds(ch * CHUNK + off, 16)]
                        v = plsc.load_gather(tab, [iv])
                        if subtract:
                            pos = (jnp.int32(ch * CHUNK) + off
                                   + lax.broadcasted_iota(jnp.int32, (16,), 0))
                            mv = lax.shift_right_logical(pos, 5)
                            v = v - plsc.load_gather(nx_v, [mv])
                        buf[pl.ds(off, 16)] = v
                    return 0
                lax.fori_loop(0, CHUNK // 128, body, 0)
                pending[s] = pltpu.async_copy(
                    buf, out_hbm.at[b, cc, pl.ds(ch * CHUNK, CHUNK)],
                    osems[s])

        @pl.when(j < 3)
        def _():
            pltpu.sync_copy(xyzt_hbm.at[b, j], xtab_v)
            pltpu.sync_copy(nxt_hbm.at[b, j], nx_v)
            gather_channel(j, xtab_v, True)
            for s in range(2):
                if pending[s] is not None:
                    pending[s].wait()
                    pending[s] = None

        pltpu.sync_copy(feat_hbm.at[b, j * NCH], tabs[0])
        for t in range(NCH):
            c = j * NCH + t
            if t + 1 < NCH:
                tcp = pltpu.async_copy(
                    feat_hbm.at[b, c + 1], tabs[(t + 1) % 2], tsem)
            gather_channel(3 + c, tabs[t % 2], False)
            if t + 1 < NCH:
                tcp.wait()
        for p in pending:
            if p is not None:
                p.wait()

    return gather_kernel


def kernel(xyz, new_xyz, features):
    B, N, _ = xyz.shape
    M = new_xyz.shape[1]
    C = features.shape[1]
    pad_q = jnp.zeros((B, M, 5), jnp.float32)
    qpad = jnp.concatenate([new_xyz, pad_q], axis=-1)
    xt = jnp.transpose(xyz, (0, 2, 1))
    pad_x = jnp.zeros((B, 5, N), jnp.float32)
    xtpad = jnp.concatenate([xt, pad_x], axis=1)
    idx = _knn(qpad, xtpad)
    nxt = jnp.transpose(new_xyz, (0, 2, 1))
    idxf = idx.reshape(B, M * NSAMPLE)
    out = _make_gather(B, C, N, M)(features, xt, nxt, idxf)
    return out.reshape(B, C + 3, M, NSAMPLE)


# speedup vs baseline: 7.9700x; 1.0331x over previous
import functools

import jax
import jax.numpy as jnp
from jax import lax
from jax.experimental import pallas as pl
from jax.experimental.pallas import tpu as pltpu
from jax.experimental.pallas import tpu_sc as plsc

_INTERPRET = False

NSAMPLE = 32
QB = 512


def _knn_body(q_ref, xt_ref, idx_ref):
    q = q_ref[0]
    xt = xt_ref[0]
    n = xt.shape[1]
    qb = q.astype(jnp.bfloat16).astype(jnp.float32)
    xb = xt.astype(jnp.bfloat16).astype(jnp.float32)
    ab = ((qb[:, 0:1] * xb[0:1, :] + qb[:, 1:2] * xb[1:2, :])
          + qb[:, 2:3] * xb[2:3, :])
    a2 = (q[:, 0:1] * q[:, 0:1] + q[:, 1:2] * q[:, 1:2]) + q[:, 2:3] * q[:, 2:3]
    b2 = (xt[0:1, :] * xt[0:1, :] + xt[1:2, :] * xt[1:2, :]) + xt[2:3, :] * xt[2:3, :]
    d2 = a2 - 2.0 * ab + b2
    iota = lax.broadcasted_iota(jnp.int32, d2.shape, 1)
    inf = jnp.float32(jnp.inf)
    big = jnp.int32(n)
    for k in range(NSAMPLE):
        m = jnp.min(d2, axis=1, keepdims=True)
        cand = jnp.where(d2 == m, iota, big)
        amin = jnp.min(cand, axis=1, keepdims=True)
        idx_ref[0, :, k:k + 1] = amin
        d2 = jnp.where(cand == amin, inf, d2)


def _knn(qpad, xtpad):
    B, M, _ = qpad.shape
    N = xtpad.shape[2]
    return pl.pallas_call(
        _knn_body,
        grid=(B, M // QB),
        in_specs=[
            pl.BlockSpec((1, QB, 8), lambda b, m: (b, m, 0)),
            pl.BlockSpec((1, 8, N), lambda b, m: (b, 0, 0)),
        ],
        out_specs=pl.BlockSpec((1, QB, NSAMPLE), lambda b, m: (b, m, 0)),
        out_shape=jax.ShapeDtypeStruct((B, M, NSAMPLE), jnp.int32),
        interpret=_INTERPRET,
    )(qpad, xtpad)


def _make_gather(B, C, N, M):
    K = NSAMPLE
    MK = M * K
    CHUNK = min(16384, MK)
    ROWS = CHUNK // K
    NCH = C // 16
    mesh = plsc.VectorSubcoreMesh(
        core_axis_name="c", subcore_axis_name="s", num_cores=2,
        num_subcores=16)

    @functools.partial(
        pl.kernel,
        out_type=jax.ShapeDtypeStruct((B, C + 3, MK), jnp.float32),
        mesh=mesh,
        scratch_types=[
            pltpu.VMEM((MK,), jnp.int32),
            pltpu.VMEM((N,), jnp.float32),
            pltpu.VMEM((N,), jnp.float32),
            pltpu.VMEM((CHUNK,), jnp.float32),
            pltpu.VMEM((CHUNK,), jnp.float32),
            pltpu.VMEM((M,), jnp.float32),
            pltpu.VMEM((N,), jnp.float32),
            pltpu.SemaphoreType.DMA,
            pltpu.SemaphoreType.DMA,
            pltpu.SemaphoreType.DMA,
        ],
        compiler_params=pltpu.CompilerParams(needs_layout_passes=False),
        interpret=_INTERPRET,
    )
    def gather_kernel(feat_hbm, xyzt_hbm, nxt_hbm, idx_hbm, out_hbm,
                      idx_v, tab0_v, tab1_v, out0_v, out1_v, nx_v, xtab_v,
                      sem0, sem1, tsem):
        osems = (sem0, sem1)
        tabs = (tab0_v, tab1_v)
        outs = (out0_v, out1_v)
        wid = lax.axis_index("s") * 2 + lax.axis_index("c")
        b = wid // 16
        j = wid % 16
        pltpu.sync_copy(idx_hbm.at[b], idx_v)
        pending = [None, None]
        nch_issued = [0]

        def gather_channel(cc, tab, subtract):
            for ch in range(MK // CHUNK):
                s = nch_issued[0] % 2
                nch_issued[0] += 1
                if pending[s] is not None:
                    pending[s].wait()
                buf = outs[s]

                def body(i, _, ch=ch, buf=buf, tab=tab):
                    for u in range(8):
                        off = i * 128 + u * 16
                        iv = idx_v[pl.ds(ch * CHUNK + off, 16)]
                        v = plsc.load_gather(tab, [iv])
                        if subtract:
                            pos = (jnp.int32(ch * CHUNK) + off
                                   + lax.broadcasted_iota(jnp.int32, (16,), 0))
                            mv = lax.shift_right_logical(pos, 5)
                            v = v - plsc.load_gather(nx_v, [mv])
                        buf[pl.ds(off, 16)] = v
                    return 0
                lax.fori_loop(0, CHUNK // 128, body, 0)
                pending[s] = pltpu.async_copy(
                    buf, out_hbm.at[b, cc, pl.ds(ch * CHUNK, CHUNK)],
                    osems[s])

        @pl.when(j < 3)
        def _():
            pltpu.sync_copy(xyzt_hbm.at[b, j], xtab_v)
            pltpu.sync_copy(nxt_hbm.at[b, j], nx_v)
            gather_channel(j, xtab_v, True)
            for s in range(2):
                if pending[s] is not None:
                    pending[s].wait()
                    pending[s] = None

        pltpu.sync_copy(feat_hbm.at[b, j * NCH], tabs[0])
        for t in range(NCH):
            c = j * NCH + t
            if t + 1 < NCH:
                tcp = pltpu.async_copy(
                    feat_hbm.at[b, c + 1], tabs[(t + 1) % 2], tsem)
            gather_channel(3 + c, tabs[t % 2], False)
            if t + 1 < NCH:
                tcp.wait()
        for p in pending:
            if p is not None:
                p.wait()

    return gather_kernel


def kernel(xyz, new_xyz, features):
    B, N, _ = xyz.shape
    M = new_xyz.shape[1]
    C = features.shape[1]
    pad_q = jnp.zeros((B, M, 5), jnp.float32)
    qpad = jnp.concatenate([new_xyz, pad_q], axis=-1)
    xt = jnp.transpose(xyz, (0, 2, 1))
    pad_x = jnp.zeros((B, 5, N), jnp.float32)
    xtpad = jnp.concatenate([xt, pad_x], axis=1)
    nxt = jnp.transpose(new_xyz, (0, 2, 1))
    S = 4
    MS = M // S
    gather = _make_gather(B, C, N, MS)
    parts = []
    for s in range(S):
        qs = lax.slice_in_dim(qpad, s * MS, (s + 1) * MS, axis=1)
        idx = _knn(qs, xtpad)
        nxs = lax.slice_in_dim(nxt, s * MS, (s + 1) * MS, axis=2)
        o = gather(features, xt, nxs, idx.reshape(B, MS * NSAMPLE))
        parts.append(o.reshape(B, C + 3, MS, NSAMPLE))
    return jnp.concatenate(parts, axis=2)


# speedup vs baseline: 8.0445x; 1.0093x over previous
import functools

import jax
import jax.numpy as jnp
from jax import lax
from jax.experimental import pallas as pl
from jax.experimental.pallas import tpu as pltpu
from jax.experimental.pallas import tpu_sc as plsc

_INTERPRET = False

NSAMPLE = 32
QB = 512


def _knn_body(q_ref, xt_ref, idx_ref):
    q = q_ref[0]
    xt = xt_ref[0]
    n = xt.shape[1]
    qb = q.astype(jnp.bfloat16).astype(jnp.float32)
    xb = xt.astype(jnp.bfloat16).astype(jnp.float32)
    ab = ((qb[:, 0:1] * xb[0:1, :] + qb[:, 1:2] * xb[1:2, :])
          + qb[:, 2:3] * xb[2:3, :])
    a2 = (q[:, 0:1] * q[:, 0:1] + q[:, 1:2] * q[:, 1:2]) + q[:, 2:3] * q[:, 2:3]
    b2 = (xt[0:1, :] * xt[0:1, :] + xt[1:2, :] * xt[1:2, :]) + xt[2:3, :] * xt[2:3, :]
    d2 = a2 - 2.0 * ab + b2
    iota = lax.broadcasted_iota(jnp.int32, d2.shape, 1)
    inf = jnp.float32(jnp.inf)
    big = jnp.int32(n)
    for k in range(NSAMPLE):
        m = jnp.min(d2, axis=1, keepdims=True)
        cand = jnp.where(d2 == m, iota, big)
        amin = jnp.min(cand, axis=1, keepdims=True)
        idx_ref[0, :, k:k + 1] = amin
        d2 = jnp.where(cand == amin, inf, d2)


def _knn(qpad, xtpad):
    B, M, _ = qpad.shape
    N = xtpad.shape[2]
    return pl.pallas_call(
        _knn_body,
        grid=(B, M // QB),
        in_specs=[
            pl.BlockSpec((1, QB, 8), lambda b, m: (b, m, 0)),
            pl.BlockSpec((1, 8, N), lambda b, m: (b, 0, 0)),
        ],
        out_specs=pl.BlockSpec((1, QB, NSAMPLE), lambda b, m: (b, m, 0)),
        out_shape=jax.ShapeDtypeStruct((B, M, NSAMPLE), jnp.int32),
        interpret=_INTERPRET,
    )(qpad, xtpad)


def _make_gather(B, C, N, M):
    K = NSAMPLE
    MK = M * K
    CHUNK = min(16384, MK)
    ROWS = CHUNK // K
    NCH = C // 16
    mesh = plsc.VectorSubcoreMesh(
        core_axis_name="c", subcore_axis_name="s", num_cores=2,
        num_subcores=16)

    @functools.partial(
        pl.kernel,
        out_type=jax.ShapeDtypeStruct((B, C + 3, MK), jnp.float32),
        mesh=mesh,
        scratch_types=[
            pltpu.VMEM((MK,), jnp.int32),
            pltpu.VMEM((N,), jnp.float32),
            pltpu.VMEM((N,), jnp.float32),
            pltpu.VMEM((CHUNK,), jnp.float32),
            pltpu.VMEM((CHUNK,), jnp.float32),
            pltpu.VMEM((M,), jnp.float32),
            pltpu.VMEM((N,), jnp.float32),
            pltpu.SemaphoreType.DMA,
            pltpu.SemaphoreType.DMA,
            pltpu.SemaphoreType.DMA,
        ],
        compiler_params=pltpu.CompilerParams(needs_layout_passes=False),
        interpret=_INTERPRET,
    )
    def gather_kernel(feat_hbm, xyzt_hbm, nxt_hbm, idx_hbm, out_hbm,
                      idx_v, tab0_v, tab1_v, out0_v, out1_v, nx_v, xtab_v,
                      sem0, sem1, tsem):
        osems = (sem0, sem1)
        tabs = (tab0_v, tab1_v)
        outs = (out0_v, out1_v)
        wid = lax.axis_index("s") * 2 + lax.axis_index("c")
        b = wid // 16
        j = wid % 16
        pltpu.sync_copy(idx_hbm.at[b], idx_v)
        pending = [None, None]
        nch_issued = [0]

        def gather_channel(cc, tab, subtract):
            for ch in range(MK // CHUNK):
                s = nch_issued[0] % 2
                nch_issued[0] += 1
                if pending[s] is not None:
                    pending[s].wait()
                buf = outs[s]

                def body(i, _, ch=ch, buf=buf, tab=tab):
                    for u in range(8):
                        off = i * 128 + u * 16
                        iv = idx_v[pl.ds(ch * CHUNK + off, 16)]
                        v = plsc.load_gather(tab, [iv])
                        if subtract:
                            pos = (jnp.int32(ch * CHUNK) + off
                                   + lax.broadcasted_iota(jnp.int32, (16,), 0))
                            mv = lax.shift_right_logical(pos, 5)
                            v = v - plsc.load_gather(nx_v, [mv])
                        buf[pl.ds(off, 16)] = v
                    return 0
                lax.fori_loop(0, CHUNK // 128, body, 0)
                pending[s] = pltpu.async_copy(
                    buf, out_hbm.at[b, cc, pl.ds(ch * CHUNK, CHUNK)],
                    osems[s])

        @pl.when(j < 3)
        def _():
            pltpu.sync_copy(xyzt_hbm.at[b, j], xtab_v)
            pltpu.sync_copy(nxt_hbm.at[b, j], nx_v)
            gather_channel(j, xtab_v, True)
            for s in range(2):
                if pending[s] is not None:
                    pending[s].wait()
                    pending[s] = None

        pltpu.sync_copy(feat_hbm.at[b, j * NCH], tabs[0])
        for t in range(NCH):
            c = j * NCH + t
            if t + 1 < NCH:
                tcp = pltpu.async_copy(
                    feat_hbm.at[b, c + 1], tabs[(t + 1) % 2], tsem)
            gather_channel(3 + c, tabs[t % 2], False)
            if t + 1 < NCH:
                tcp.wait()
        for p in pending:
            if p is not None:
                p.wait()

    return gather_kernel


def kernel(xyz, new_xyz, features):
    B, N, _ = xyz.shape
    M = new_xyz.shape[1]
    C = features.shape[1]
    pad_q = jnp.zeros((B, M, 5), jnp.float32)
    qpad = jnp.concatenate([new_xyz, pad_q], axis=-1)
    xt = jnp.transpose(xyz, (0, 2, 1))
    pad_x = jnp.zeros((B, 5, N), jnp.float32)
    xtpad = jnp.concatenate([xt, pad_x], axis=1)
    nxt = jnp.transpose(new_xyz, (0, 2, 1))
    S = 2
    MS = M // S
    gather = _make_gather(B, C, N, MS)
    parts = []
    for s in range(S):
        qs = lax.slice_in_dim(qpad, s * MS, (s + 1) * MS, axis=1)
        idx = _knn(qs, xtpad)
        nxs = lax.slice_in_dim(nxt, s * MS, (s + 1) * MS, axis=2)
        o = gather(features, xt, nxs, idx.reshape(B, MS * NSAMPLE))
        parts.append(o.reshape(B, C + 3, MS, NSAMPLE))
    return jnp.concatenate(parts, axis=2)
